# baseline (speedup 1.0000x reference)
import functools
import math

import jax
import jax.numpy as jnp
from jax import lax
from jax.experimental import pallas as pl
from jax.experimental.pallas import tpu as pltpu

_NUM_TOKENS = 4096
_MODEL_DIM = 4096
_NUM_EXPERTS = 64
_CAPACITY = 2 * int(math.ceil(_NUM_TOKENS / _NUM_EXPERTS))

_T1 = 512
_T2 = 256
_NB1 = _NUM_TOKENS // _T1
_NB2 = _NUM_TOKENS // _T2

_EPS = float(jnp.finfo(jnp.float32).eps)


def _router_kernel(x_ref, wg_ref,
                   idx1_ref, idx2_ref, loc1_ref, loc2_ref, g1_ref, g2_ref,
                   cnt1_ref, laux_ref,
                   carry1, carry2, sgates):
    i = pl.program_id(0)

    @pl.when(i == 0)
    def _init():
        carry1[...] = jnp.zeros_like(carry1)
        carry2[...] = jnp.zeros_like(carry2)
        sgates[...] = jnp.zeros_like(sgates)

    x = x_ref[...]
    wg = wg_ref[...]
    logits = jnp.dot(x, wg, preferred_element_type=jnp.float32)

    m = jnp.max(logits, axis=1, keepdims=True)
    e = jnp.exp(logits - m)
    z = jnp.sum(e, axis=1, keepdims=True)
    gates = e / z

    iota_e = lax.broadcasted_iota(jnp.int32, (_T1, _NUM_EXPERTS), 1)

    is1 = logits == m
    idx1 = jnp.min(jnp.where(is1, iota_e, _NUM_EXPERTS), axis=1, keepdims=True)
    mask1 = (iota_e == idx1).astype(jnp.float32)

    logits2 = jnp.where(iota_e == idx1, -jnp.inf, logits)
    m2 = jnp.max(logits2, axis=1, keepdims=True)
    is2 = logits2 == m2
    idx2 = jnp.min(jnp.where(is2, iota_e, _NUM_EXPERTS), axis=1, keepdims=True)
    mask2 = (iota_e == idx2).astype(jnp.float32)

    r = lax.broadcasted_iota(jnp.int32, (_T1, _T1), 0)
    c = lax.broadcasted_iota(jnp.int32, (_T1, _T1), 1)
    tril = (r > c).astype(jnp.float32)
    pre1 = jnp.dot(tril, mask1, preferred_element_type=jnp.float32) + carry1[...]
    pre2 = jnp.dot(tril, mask2, preferred_element_type=jnp.float32) + carry2[...]

    loc1 = jnp.sum(pre1 * mask1, axis=1, keepdims=True)
    loc2 = jnp.sum(pre2 * mask2, axis=1, keepdims=True)
    g1 = jnp.sum(gates * mask1, axis=1, keepdims=True)
    g2 = jnp.sum(gates * mask2, axis=1, keepdims=True)

    idx1_ref[...] = idx1.astype(jnp.float32)
    idx2_ref[...] = idx2.astype(jnp.float32)
    loc1_ref[...] = loc1
    loc2_ref[...] = loc2
    g1_ref[...] = g1
    g2_ref[...] = g2

    carry1[...] = carry1[...] + jnp.sum(mask1, axis=0, keepdims=True)
    carry2[...] = carry2[...] + jnp.sum(mask2, axis=0, keepdims=True)
    sgates[...] = sgates[...] + jnp.sum(gates, axis=0, keepdims=True)

    @pl.when(i == _NB1 - 1)
    def _fin():
        cnt1 = carry1[...]
        cnt1_ref[...] = cnt1
        scale = jnp.float32(_NUM_EXPERTS / (_NUM_TOKENS * _NUM_TOKENS))
        laux_ref[...] = jnp.sum(cnt1 * sgates[...], keepdims=True).reshape(1, 1) * scale


def _expand_kernel(idx1_ref, idx2_ref, loc1_ref, loc2_ref, g1_ref, g2_ref,
                   cnt1_ref, comb_ref, mask_ref):
    idx1 = idx1_ref[...]
    idx2 = idx2_ref[...]
    loc1 = loc1_ref[...]
    loc2p = loc2_ref[...]
    g1 = g1_ref[...]
    g2 = g2_ref[...]
    cnt1 = cnt1_ref[...]

    iota_e = lax.broadcasted_iota(jnp.int32, (_T2, _NUM_EXPERTS), 1).astype(jnp.float32)
    onehot2 = (iota_e == idx2).astype(jnp.float32)
    loc2 = loc2p + jnp.sum(cnt1 * onehot2, axis=1, keepdims=True)

    cap = jnp.float32(_CAPACITY)
    w1 = jnp.where(loc1 < cap, g1, 0.0)
    w2 = jnp.where(loc2 < cap, g2, 0.0)
    denom = jnp.maximum(w1 + w2, _EPS)
    w1 = w1 / denom
    w2 = w2 / denom

    pos1 = idx1 * cap + loc1
    pos2 = idx2 * cap + loc2
    iota_f = lax.broadcasted_iota(
        jnp.int32, (_T2, _NUM_EXPERTS * _CAPACITY), 1).astype(jnp.float32)
    comb = jnp.where(iota_f == pos1, w1, 0.0) + jnp.where(iota_f == pos2, w2, 0.0)
    comb_ref[...] = comb
    mask_ref[...] = comb != 0.0


@jax.jit
def kernel(input, wg):
    n, d = input.shape
    e = wg.shape[1]
    ec = _NUM_EXPERTS * _CAPACITY

    meta_spec = pl.BlockSpec((_T1, 1), lambda i: (i, 0))
    router = pl.pallas_call(
        _router_kernel,
        grid=(_NB1,),
        in_specs=[
            pl.BlockSpec((_T1, _MODEL_DIM), lambda i: (i, 0)),
            pl.BlockSpec((_MODEL_DIM, _NUM_EXPERTS), lambda i: (0, 0)),
        ],
        out_specs=[
            meta_spec, meta_spec, meta_spec, meta_spec, meta_spec, meta_spec,
            pl.BlockSpec((1, _NUM_EXPERTS), lambda i: (0, 0)),
            pl.BlockSpec((1, 1), lambda i: (0, 0)),
        ],
        out_shape=[
            jax.ShapeDtypeStruct((n, 1), jnp.float32),
            jax.ShapeDtypeStruct((n, 1), jnp.float32),
            jax.ShapeDtypeStruct((n, 1), jnp.float32),
            jax.ShapeDtypeStruct((n, 1), jnp.float32),
            jax.ShapeDtypeStruct((n, 1), jnp.float32),
            jax.ShapeDtypeStruct((n, 1), jnp.float32),
            jax.ShapeDtypeStruct((1, _NUM_EXPERTS), jnp.float32),
            jax.ShapeDtypeStruct((1, 1), jnp.float32),
        ],
        scratch_shapes=[
            pltpu.VMEM((1, _NUM_EXPERTS), jnp.float32),
            pltpu.VMEM((1, _NUM_EXPERTS), jnp.float32),
            pltpu.VMEM((1, _NUM_EXPERTS), jnp.float32),
        ],
    )
    idx1, idx2, loc1, loc2, g1, g2, cnt1, laux = router(input, wg)

    meta_spec2 = pl.BlockSpec((_T2, 1), lambda i: (i, 0))
    expand = pl.pallas_call(
        _expand_kernel,
        grid=(_NB2,),
        in_specs=[
            meta_spec2, meta_spec2, meta_spec2, meta_spec2, meta_spec2, meta_spec2,
            pl.BlockSpec((1, _NUM_EXPERTS), lambda i: (0, 0)),
        ],
        out_specs=[
            pl.BlockSpec((_T2, ec), lambda i: (i, 0)),
            pl.BlockSpec((_T2, ec), lambda i: (i, 0)),
        ],
        out_shape=[
            jax.ShapeDtypeStruct((n, ec), jnp.float32),
            jax.ShapeDtypeStruct((n, ec), jnp.bool_),
        ],
    )
    comb, mask = expand(idx1, idx2, loc1, loc2, g1, g2, cnt1)

    combine_weights = comb.reshape(n, e, _CAPACITY)
    dispatch_mask = mask.reshape(n, e, _CAPACITY)
    l_aux = laux.reshape(())
    return (l_aux, combine_weights, dispatch_mask)


# speedup vs baseline: 1.8232x; 1.8232x over previous
import functools
import math

import jax
import jax.numpy as jnp
from jax import lax
from jax.experimental import pallas as pl
from jax.experimental.pallas import tpu as pltpu

_NUM_TOKENS = 4096
_MODEL_DIM = 4096
_NUM_EXPERTS = 64
_CAPACITY = 2 * int(math.ceil(_NUM_TOKENS / _NUM_EXPERTS))

_T1 = 512
_T2 = 256
_NB1 = _NUM_TOKENS // _T1
_NB2 = _NUM_TOKENS // _T2

_EPS = float(jnp.finfo(jnp.float32).eps)


def _router_kernel(x_ref, wg_ref,
                   idx1_ref, idx2_ref, loc1_ref, loc2_ref, g1_ref, g2_ref,
                   cnt1_ref, laux_ref,
                   carry1, carry2, sgates):
    i = pl.program_id(0)

    @pl.when(i == 0)
    def _init():
        carry1[...] = jnp.zeros_like(carry1)
        carry2[...] = jnp.zeros_like(carry2)
        sgates[...] = jnp.zeros_like(sgates)

    x = x_ref[...]
    wg = wg_ref[...]
    logits = jnp.dot(x, wg, preferred_element_type=jnp.float32)

    m = jnp.max(logits, axis=1, keepdims=True)
    e = jnp.exp(logits - m)
    z = jnp.sum(e, axis=1, keepdims=True)
    gates = e / z

    iota_e = lax.broadcasted_iota(jnp.int32, (_T1, _NUM_EXPERTS), 1)

    is1 = logits == m
    idx1 = jnp.min(jnp.where(is1, iota_e, _NUM_EXPERTS), axis=1, keepdims=True)
    mask1 = (iota_e == idx1).astype(jnp.float32)

    logits2 = jnp.where(iota_e == idx1, -jnp.inf, logits)
    m2 = jnp.max(logits2, axis=1, keepdims=True)
    is2 = logits2 == m2
    idx2 = jnp.min(jnp.where(is2, iota_e, _NUM_EXPERTS), axis=1, keepdims=True)
    mask2 = (iota_e == idx2).astype(jnp.float32)

    r = lax.broadcasted_iota(jnp.int32, (_T1, _T1), 0)
    c = lax.broadcasted_iota(jnp.int32, (_T1, _T1), 1)
    tril = (r > c).astype(jnp.float32)
    pre1 = jnp.dot(tril, mask1, preferred_element_type=jnp.float32) + carry1[...]
    pre2 = jnp.dot(tril, mask2, preferred_element_type=jnp.float32) + carry2[...]

    loc1 = jnp.sum(pre1 * mask1, axis=1, keepdims=True)
    loc2 = jnp.sum(pre2 * mask2, axis=1, keepdims=True)
    g1 = jnp.sum(gates * mask1, axis=1, keepdims=True)
    g2 = jnp.sum(gates * mask2, axis=1, keepdims=True)

    idx1_ref[...] = idx1.astype(jnp.float32)
    idx2_ref[...] = idx2.astype(jnp.float32)
    loc1_ref[...] = loc1
    loc2_ref[...] = loc2
    g1_ref[...] = g1
    g2_ref[...] = g2

    carry1[...] = carry1[...] + jnp.sum(mask1, axis=0, keepdims=True)
    carry2[...] = carry2[...] + jnp.sum(mask2, axis=0, keepdims=True)
    sgates[...] = sgates[...] + jnp.sum(gates, axis=0, keepdims=True)

    @pl.when(i == _NB1 - 1)
    def _fin():
        cnt1 = carry1[...]
        cnt1_ref[...] = cnt1
        scale = jnp.float32(_NUM_EXPERTS / (_NUM_TOKENS * _NUM_TOKENS))
        laux_ref[...] = jnp.sum(cnt1 * sgates[...], keepdims=True).reshape(1, 1) * scale


def _expand_kernel(idx1_ref, idx2_ref, loc1_ref, loc2_ref, g1_ref, g2_ref,
                   cnt1_ref, comb_ref, mask_ref):
    idx1 = idx1_ref[...]
    idx2 = idx2_ref[...]
    loc1 = loc1_ref[...]
    loc2p = loc2_ref[...]
    g1 = g1_ref[...]
    g2 = g2_ref[...]
    cnt1 = cnt1_ref[...]

    iota_e = lax.broadcasted_iota(jnp.int32, (_T2, _NUM_EXPERTS), 1).astype(jnp.float32)
    onehot2 = (iota_e == idx2).astype(jnp.float32)
    loc2 = loc2p + jnp.sum(cnt1 * onehot2, axis=1, keepdims=True)

    cap = jnp.float32(_CAPACITY)
    w1 = jnp.where(loc1 < cap, g1, 0.0)
    w2 = jnp.where(loc2 < cap, g2, 0.0)
    denom = jnp.maximum(w1 + w2, _EPS)
    w1 = w1 / denom
    w2 = w2 / denom

    onehot1 = (iota_e == idx1).astype(jnp.float32)
    iota_c = lax.broadcasted_iota(jnp.int32, (_T2, _CAPACITY), 1).astype(jnp.float32)
    c1 = (iota_c == loc1).astype(jnp.float32)
    c2 = (iota_c == loc2).astype(jnp.float32)
    comb = ((onehot1 * w1)[:, :, None] * c1[:, None, :]
            + (onehot2 * w2)[:, :, None] * c2[:, None, :])
    comb_ref[...] = comb
    mask_ref[...] = comb != 0.0


@jax.jit
def kernel(input, wg):
    n, d = input.shape
    e = wg.shape[1]
    ec = _NUM_EXPERTS * _CAPACITY

    meta_spec = pl.BlockSpec((_T1, 1), lambda i: (i, 0))
    router = pl.pallas_call(
        _router_kernel,
        grid=(_NB1,),
        in_specs=[
            pl.BlockSpec((_T1, _MODEL_DIM), lambda i: (i, 0)),
            pl.BlockSpec((_MODEL_DIM, _NUM_EXPERTS), lambda i: (0, 0)),
        ],
        out_specs=[
            meta_spec, meta_spec, meta_spec, meta_spec, meta_spec, meta_spec,
            pl.BlockSpec((1, _NUM_EXPERTS), lambda i: (0, 0)),
            pl.BlockSpec((1, 1), lambda i: (0, 0)),
        ],
        out_shape=[
            jax.ShapeDtypeStruct((n, 1), jnp.float32),
            jax.ShapeDtypeStruct((n, 1), jnp.float32),
            jax.ShapeDtypeStruct((n, 1), jnp.float32),
            jax.ShapeDtypeStruct((n, 1), jnp.float32),
            jax.ShapeDtypeStruct((n, 1), jnp.float32),
            jax.ShapeDtypeStruct((n, 1), jnp.float32),
            jax.ShapeDtypeStruct((1, _NUM_EXPERTS), jnp.float32),
            jax.ShapeDtypeStruct((1, 1), jnp.float32),
        ],
        scratch_shapes=[
            pltpu.VMEM((1, _NUM_EXPERTS), jnp.float32),
            pltpu.VMEM((1, _NUM_EXPERTS), jnp.float32),
            pltpu.VMEM((1, _NUM_EXPERTS), jnp.float32),
        ],
    )
    idx1, idx2, loc1, loc2, g1, g2, cnt1, laux = router(input, wg)

    meta_spec2 = pl.BlockSpec((_T2, 1), lambda i: (i, 0))
    expand = pl.pallas_call(
        _expand_kernel,
        grid=(_NB2,),
        in_specs=[
            meta_spec2, meta_spec2, meta_spec2, meta_spec2, meta_spec2, meta_spec2,
            pl.BlockSpec((1, _NUM_EXPERTS), lambda i: (0, 0)),
        ],
        out_specs=[
            pl.BlockSpec((_T2, _NUM_EXPERTS, _CAPACITY), lambda i: (i, 0, 0)),
            pl.BlockSpec((_T2, _NUM_EXPERTS, _CAPACITY), lambda i: (i, 0, 0)),
        ],
        out_shape=[
            jax.ShapeDtypeStruct((n, _NUM_EXPERTS, _CAPACITY), jnp.float32),
            jax.ShapeDtypeStruct((n, _NUM_EXPERTS, _CAPACITY), jnp.bool_),
        ],
    )
    combine_weights, dispatch_mask = expand(idx1, idx2, loc1, loc2, g1, g2, cnt1)
    l_aux = laux.reshape(())
    return (l_aux, combine_weights, dispatch_mask)


# speedup vs baseline: 1.8289x; 1.0031x over previous
import functools
import math

import jax
import jax.numpy as jnp
from jax import lax
from jax.experimental import pallas as pl
from jax.experimental.pallas import tpu as pltpu

_NUM_TOKENS = 4096
_MODEL_DIM = 4096
_NUM_EXPERTS = 64
_CAPACITY = 2 * int(math.ceil(_NUM_TOKENS / _NUM_EXPERTS))

_T1 = 512
_T2 = 256
_NB1 = _NUM_TOKENS // _T1
_NB2 = _NUM_TOKENS // _T2

_EPS = float(jnp.finfo(jnp.float32).eps)


def _router_kernel(x_ref, wg_ref,
                   idx1_ref, idx2_ref, loc1_ref, loc2_ref, g1_ref, g2_ref,
                   cnt1_ref, laux_ref,
                   carry1, carry2, sgates):
    i = pl.program_id(0)

    @pl.when(i == 0)
    def _init():
        carry1[...] = jnp.zeros_like(carry1)
        carry2[...] = jnp.zeros_like(carry2)
        sgates[...] = jnp.zeros_like(sgates)

    x = x_ref[...]
    wg = wg_ref[...]
    logits = jnp.dot(x, wg, preferred_element_type=jnp.float32)

    m = jnp.max(logits, axis=1, keepdims=True)
    e = jnp.exp(logits - m)
    z = jnp.sum(e, axis=1, keepdims=True)
    gates = e / z

    iota_e = lax.broadcasted_iota(jnp.int32, (_T1, _NUM_EXPERTS), 1)

    is1 = logits == m
    idx1 = jnp.min(jnp.where(is1, iota_e, _NUM_EXPERTS), axis=1, keepdims=True)
    mask1 = (iota_e == idx1).astype(jnp.float32)

    logits2 = jnp.where(iota_e == idx1, -jnp.inf, logits)
    m2 = jnp.max(logits2, axis=1, keepdims=True)
    is2 = logits2 == m2
    idx2 = jnp.min(jnp.where(is2, iota_e, _NUM_EXPERTS), axis=1, keepdims=True)
    mask2 = (iota_e == idx2).astype(jnp.float32)

    r = lax.broadcasted_iota(jnp.int32, (_T1, _T1), 0)
    c = lax.broadcasted_iota(jnp.int32, (_T1, _T1), 1)
    tril = (r > c).astype(jnp.float32)
    pre1 = jnp.dot(tril, mask1, preferred_element_type=jnp.float32) + carry1[...]
    pre2 = jnp.dot(tril, mask2, preferred_element_type=jnp.float32) + carry2[...]

    loc1 = jnp.sum(pre1 * mask1, axis=1, keepdims=True)
    loc2 = jnp.sum(pre2 * mask2, axis=1, keepdims=True)
    g1 = jnp.sum(gates * mask1, axis=1, keepdims=True)
    g2 = jnp.sum(gates * mask2, axis=1, keepdims=True)

    idx1_ref[...] = idx1.astype(jnp.float32)
    idx2_ref[...] = idx2.astype(jnp.float32)
    loc1_ref[...] = loc1
    loc2_ref[...] = loc2
    g1_ref[...] = g1
    g2_ref[...] = g2

    carry1[...] = carry1[...] + jnp.sum(mask1, axis=0, keepdims=True)
    carry2[...] = carry2[...] + jnp.sum(mask2, axis=0, keepdims=True)
    sgates[...] = sgates[...] + jnp.sum(gates, axis=0, keepdims=True)

    @pl.when(i == _NB1 - 1)
    def _fin():
        cnt1 = carry1[...]
        cnt1_ref[...] = cnt1
        scale = jnp.float32(_NUM_EXPERTS / (_NUM_TOKENS * _NUM_TOKENS))
        laux_ref[...] = jnp.sum(cnt1 * sgates[...], keepdims=True).reshape(1, 1) * scale


def _expand_kernel(idx1_ref, idx2_ref, loc1_ref, loc2_ref, g1_ref, g2_ref,
                   cnt1_ref, comb_ref, mask_ref):
    idx1 = idx1_ref[...]
    idx2 = idx2_ref[...]
    loc1 = loc1_ref[...]
    loc2p = loc2_ref[...]
    g1 = g1_ref[...]
    g2 = g2_ref[...]
    cnt1 = cnt1_ref[...]

    iota_e = lax.broadcasted_iota(jnp.int32, (_T2, _NUM_EXPERTS), 1).astype(jnp.float32)
    onehot2 = (iota_e == idx2).astype(jnp.float32)
    loc2 = loc2p + jnp.sum(cnt1 * onehot2, axis=1, keepdims=True)

    cap = jnp.float32(_CAPACITY)
    w1 = jnp.where(loc1 < cap, g1, 0.0)
    w2 = jnp.where(loc2 < cap, g2, 0.0)
    denom = jnp.maximum(w1 + w2, _EPS)
    w1 = w1 / denom
    w2 = w2 / denom

    onehot1 = (iota_e == idx1).astype(jnp.float32)
    b = onehot1 * w1 + onehot2 * w2
    l = onehot1 * loc1 + onehot2 * loc2 + (1.0 - onehot1 - onehot2) * -1.0
    iota_c = lax.broadcasted_iota(
        jnp.int32, (_T2, _NUM_EXPERTS, _CAPACITY), 2).astype(jnp.float32)
    hit = iota_c == l[:, :, None]
    comb = jnp.where(hit, b[:, :, None], 0.0)
    comb_ref[...] = comb
    mask_ref[...] = comb != 0.0


@jax.jit
def kernel(input, wg):
    n, d = input.shape
    e = wg.shape[1]
    ec = _NUM_EXPERTS * _CAPACITY

    meta_spec = pl.BlockSpec((_T1, 1), lambda i: (i, 0))
    router = pl.pallas_call(
        _router_kernel,
        grid=(_NB1,),
        in_specs=[
            pl.BlockSpec((_T1, _MODEL_DIM), lambda i: (i, 0)),
            pl.BlockSpec((_MODEL_DIM, _NUM_EXPERTS), lambda i: (0, 0)),
        ],
        out_specs=[
            meta_spec, meta_spec, meta_spec, meta_spec, meta_spec, meta_spec,
            pl.BlockSpec((1, _NUM_EXPERTS), lambda i: (0, 0)),
            pl.BlockSpec((1, 1), lambda i: (0, 0)),
        ],
        out_shape=[
            jax.ShapeDtypeStruct((n, 1), jnp.float32),
            jax.ShapeDtypeStruct((n, 1), jnp.float32),
            jax.ShapeDtypeStruct((n, 1), jnp.float32),
            jax.ShapeDtypeStruct((n, 1), jnp.float32),
            jax.ShapeDtypeStruct((n, 1), jnp.float32),
            jax.ShapeDtypeStruct((n, 1), jnp.float32),
            jax.ShapeDtypeStruct((1, _NUM_EXPERTS), jnp.float32),
            jax.ShapeDtypeStruct((1, 1), jnp.float32),
        ],
        scratch_shapes=[
            pltpu.VMEM((1, _NUM_EXPERTS), jnp.float32),
            pltpu.VMEM((1, _NUM_EXPERTS), jnp.float32),
            pltpu.VMEM((1, _NUM_EXPERTS), jnp.float32),
        ],
    )
    idx1, idx2, loc1, loc2, g1, g2, cnt1, laux = router(input, wg)

    meta_spec2 = pl.BlockSpec((_T2, 1), lambda i: (i, 0))
    expand = pl.pallas_call(
        _expand_kernel,
        grid=(_NB2,),
        in_specs=[
            meta_spec2, meta_spec2, meta_spec2, meta_spec2, meta_spec2, meta_spec2,
            pl.BlockSpec((1, _NUM_EXPERTS), lambda i: (0, 0)),
        ],
        out_specs=[
            pl.BlockSpec((_T2, _NUM_EXPERTS, _CAPACITY), lambda i: (i, 0, 0)),
            pl.BlockSpec((_T2, _NUM_EXPERTS, _CAPACITY), lambda i: (i, 0, 0)),
        ],
        out_shape=[
            jax.ShapeDtypeStruct((n, _NUM_EXPERTS, _CAPACITY), jnp.float32),
            jax.ShapeDtypeStruct((n, _NUM_EXPERTS, _CAPACITY), jnp.bool_),
        ],
        compiler_params=pltpu.CompilerParams(
            dimension_semantics=("parallel",)),
    )
    combine_weights, dispatch_mask = expand(idx1, idx2, loc1, loc2, g1, g2, cnt1)
    l_aux = laux.reshape(())
    return (l_aux, combine_weights, dispatch_mask)


# speedup vs baseline: 1.9236x; 1.0518x over previous
import math

import jax
import jax.numpy as jnp
from jax import lax
from jax.experimental import pallas as pl
from jax.experimental.pallas import tpu as pltpu

_NUM_TOKENS = 4096
_MODEL_DIM = 4096
_NUM_EXPERTS = 64
_CAPACITY = 2 * int(math.ceil(_NUM_TOKENS / _NUM_EXPERTS))

_T1 = 256
_T2 = 256
_NB1 = _NUM_TOKENS // _T1
_NB2 = _NUM_TOKENS // _T2

_EPS = float(jnp.finfo(jnp.float32).eps)


def _gate_kernel(x_ref, wg_ref,
                 comb_ref, mask_ref, laux_ref,
                 pk1_ref, pk2_ref, g1_ref, g2_ref,
                 carry1, carry2, sgates):
    i = pl.program_id(0)

    @pl.when(i == 0)
    def _init():
        carry1[...] = jnp.zeros_like(carry1)
        carry2[...] = jnp.zeros_like(carry2)
        sgates[...] = jnp.zeros_like(sgates)

    @pl.when(i < _NB1)
    def _router():
        logits = jnp.dot(x_ref[...], wg_ref[...],
                         preferred_element_type=jnp.float32)

        m = jnp.max(logits, axis=1, keepdims=True)
        ex = jnp.exp(logits - m)
        z = jnp.sum(ex, axis=1, keepdims=True)
        gates = ex / z

        iota_e = lax.broadcasted_iota(jnp.int32, (_T1, _NUM_EXPERTS), 1)

        is1 = logits == m
        idx1 = jnp.min(jnp.where(is1, iota_e, _NUM_EXPERTS), axis=1,
                       keepdims=True)
        mask1 = (iota_e == idx1).astype(jnp.float32)

        logits2 = jnp.where(iota_e == idx1, -jnp.inf, logits)
        m2 = jnp.max(logits2, axis=1, keepdims=True)
        is2 = logits2 == m2
        idx2 = jnp.min(jnp.where(is2, iota_e, _NUM_EXPERTS), axis=1,
                       keepdims=True)
        mask2 = (iota_e == idx2).astype(jnp.float32)

        r = lax.broadcasted_iota(jnp.int32, (_T1, _T1), 0)
        c = lax.broadcasted_iota(jnp.int32, (_T1, _T1), 1)
        tril = (r > c).astype(jnp.float32)
        pre1 = jnp.dot(tril, mask1, preferred_element_type=jnp.float32)
        pre2 = jnp.dot(tril, mask2, preferred_element_type=jnp.float32)
        pre1 = pre1 + carry1[...]
        pre2 = pre2 + carry2[...]

        loc1 = jnp.sum(pre1 * mask1, axis=1, keepdims=True)
        loc2 = jnp.sum(pre2 * mask2, axis=1, keepdims=True)
        g1 = jnp.sum(gates * mask1, axis=1, keepdims=True)
        g2 = jnp.sum(gates * mask2, axis=1, keepdims=True)

        sl = pl.ds(i * _T1, _T1)
        pk1_ref[sl, :] = idx1.astype(jnp.float32) * 4096.0 + loc1
        pk2_ref[sl, :] = idx2.astype(jnp.float32) * 4096.0 + loc2
        g1_ref[sl, :] = g1
        g2_ref[sl, :] = g2

        carry1[...] = carry1[...] + jnp.sum(mask1, axis=0, keepdims=True)
        carry2[...] = carry2[...] + jnp.sum(mask2, axis=0, keepdims=True)
        sgates[...] = sgates[...] + jnp.sum(gates, axis=0, keepdims=True)

        @pl.when(i == _NB1 - 1)
        def _fin():
            cnt1 = carry1[...]
            scale = jnp.float32(_NUM_EXPERTS / (_NUM_TOKENS * _NUM_TOKENS))
            laux_ref[...] = jnp.sum(
                cnt1 * sgates[...], keepdims=True).reshape(1, 1) * scale

    @pl.when(i >= _NB1)
    def _expand():
        j = i - _NB1
        sl = pl.ds(j * _T2, _T2)
        pk1 = pk1_ref[sl, :]
        pk2 = pk2_ref[sl, :]
        g1 = g1_ref[sl, :]
        g2 = g2_ref[sl, :]
        cnt1 = carry1[...]

        idx1 = jnp.floor(pk1 * (1.0 / 4096.0))
        idx2 = jnp.floor(pk2 * (1.0 / 4096.0))
        loc1 = pk1 - idx1 * 4096.0
        loc2p = pk2 - idx2 * 4096.0

        iota_e = lax.broadcasted_iota(
            jnp.int32, (_T2, _NUM_EXPERTS), 1).astype(jnp.float32)
        onehot1 = (iota_e == idx1).astype(jnp.float32)
        onehot2 = (iota_e == idx2).astype(jnp.float32)
        loc2 = loc2p + jnp.sum(cnt1 * onehot2, axis=1, keepdims=True)

        cap = jnp.float32(_CAPACITY)
        w1 = jnp.where(loc1 < cap, g1, 0.0)
        w2 = jnp.where(loc2 < cap, g2, 0.0)
        denom = jnp.maximum(w1 + w2, _EPS)
        w1 = w1 / denom
        w2 = w2 / denom

        b = onehot1 * w1 + onehot2 * w2
        l = onehot1 * loc1 + onehot2 * loc2 + (1.0 - onehot1 - onehot2) * -1.0
        l = jnp.where(b != 0.0, l, -1.0)
        pk = l + b * 0.5
        pkb = pk[:, :, None]
        fl = jnp.floor(pkb)
        iota_c = lax.broadcasted_iota(
            jnp.int32, (_T2, _NUM_EXPERTS, _CAPACITY), 2).astype(jnp.float32)
        hit = iota_c == fl
        comb_ref[...] = jnp.where(hit, (pkb - fl) * 2.0, 0.0)
        mask_ref[...] = hit


@jax.jit
def kernel(input, wg):
    n = input.shape[0]

    gate = pl.pallas_call(
        _gate_kernel,
        grid=(_NB1 + _NB2,),
        in_specs=[
            pl.BlockSpec((_T1, _MODEL_DIM),
                         lambda i: (jnp.minimum(i, _NB1 - 1), 0)),
            pl.BlockSpec((_MODEL_DIM, _NUM_EXPERTS), lambda i: (0, 0)),
        ],
        out_specs=[
            pl.BlockSpec((_T2, _NUM_EXPERTS, _CAPACITY),
                         lambda i: (jnp.maximum(i - _NB1, 0), 0, 0)),
            pl.BlockSpec((_T2, _NUM_EXPERTS, _CAPACITY),
                         lambda i: (jnp.maximum(i - _NB1, 0), 0, 0)),
            pl.BlockSpec((1, 1), lambda i: (0, 0)),
        ],
        out_shape=[
            jax.ShapeDtypeStruct((n, _NUM_EXPERTS, _CAPACITY), jnp.float32),
            jax.ShapeDtypeStruct((n, _NUM_EXPERTS, _CAPACITY), jnp.bool_),
            jax.ShapeDtypeStruct((1, 1), jnp.float32),
        ],
        scratch_shapes=[
            pltpu.VMEM((_NUM_TOKENS, 1), jnp.float32),
            pltpu.VMEM((_NUM_TOKENS, 1), jnp.float32),
            pltpu.VMEM((_NUM_TOKENS, 1), jnp.float32),
            pltpu.VMEM((_NUM_TOKENS, 1), jnp.float32),
            pltpu.VMEM((1, _NUM_EXPERTS), jnp.float32),
            pltpu.VMEM((1, _NUM_EXPERTS), jnp.float32),
            pltpu.VMEM((1, _NUM_EXPERTS), jnp.float32),
        ],
    )
    combine_weights, dispatch_mask, laux = gate(input, wg)
    l_aux = laux.reshape(())
    return (l_aux, combine_weights, dispatch_mask)


# speedup vs baseline: 1.9953x; 1.0373x over previous
import math

import jax
import jax.numpy as jnp
from jax import lax
from jax.experimental import pallas as pl
from jax.experimental.pallas import tpu as pltpu

_NUM_TOKENS = 4096
_MODEL_DIM = 4096
_NUM_EXPERTS = 64
_CAPACITY = 2 * int(math.ceil(_NUM_TOKENS / _NUM_EXPERTS))

_T1 = 512
_T2 = 128
_NB1 = _NUM_TOKENS // _T1
_NB2 = _NUM_TOKENS // _T2

_EPS = float(jnp.finfo(jnp.float32).eps)


def _gate_kernel(x_ref, wg_ref,
                 comb_ref, mask_ref, laux_ref,
                 pk1_ref, pk2_ref, g1_ref, g2_ref,
                 carry1, carry2, sgates):
    i = pl.program_id(0)

    @pl.when(i == 0)
    def _init():
        carry1[...] = jnp.zeros_like(carry1)
        carry2[...] = jnp.zeros_like(carry2)
        sgates[...] = jnp.zeros_like(sgates)

    @pl.when(i < _NB1)
    def _router():
        logits = jnp.dot(x_ref[...], wg_ref[...],
                         preferred_element_type=jnp.float32)

        m = jnp.max(logits, axis=1, keepdims=True)
        ex = jnp.exp(logits - m)
        z = jnp.sum(ex, axis=1, keepdims=True)
        gates = ex / z

        iota_e = lax.broadcasted_iota(jnp.int32, (_T1, _NUM_EXPERTS), 1)

        is1 = logits == m
        idx1 = jnp.min(jnp.where(is1, iota_e, _NUM_EXPERTS), axis=1,
                       keepdims=True)
        mask1 = (iota_e == idx1).astype(jnp.float32)

        logits2 = jnp.where(iota_e == idx1, -jnp.inf, logits)
        m2 = jnp.max(logits2, axis=1, keepdims=True)
        is2 = logits2 == m2
        idx2 = jnp.min(jnp.where(is2, iota_e, _NUM_EXPERTS), axis=1,
                       keepdims=True)
        mask2 = (iota_e == idx2).astype(jnp.float32)

        r = lax.broadcasted_iota(jnp.int32, (_T1, _T1), 0)
        c = lax.broadcasted_iota(jnp.int32, (_T1, _T1), 1)
        tril = (r > c).astype(jnp.float32)
        pre1 = jnp.dot(tril, mask1, preferred_element_type=jnp.float32)
        pre2 = jnp.dot(tril, mask2, preferred_element_type=jnp.float32)
        pre1 = pre1 + carry1[...]
        pre2 = pre2 + carry2[...]

        loc1 = jnp.sum(pre1 * mask1, axis=1, keepdims=True)
        loc2 = jnp.sum(pre2 * mask2, axis=1, keepdims=True)
        g1 = jnp.sum(gates * mask1, axis=1, keepdims=True)
        g2 = jnp.sum(gates * mask2, axis=1, keepdims=True)

        sl = pl.ds(i * _T1, _T1)
        pk1_ref[sl, :] = idx1.astype(jnp.float32) * 4096.0 + loc1
        pk2_ref[sl, :] = idx2.astype(jnp.float32) * 4096.0 + loc2
        g1_ref[sl, :] = g1
        g2_ref[sl, :] = g2

        carry1[...] = carry1[...] + jnp.sum(mask1, axis=0, keepdims=True)
        carry2[...] = carry2[...] + jnp.sum(mask2, axis=0, keepdims=True)
        sgates[...] = sgates[...] + jnp.sum(gates, axis=0, keepdims=True)

        @pl.when(i == _NB1 - 1)
        def _fin():
            cnt1 = carry1[...]
            scale = jnp.float32(_NUM_EXPERTS / (_NUM_TOKENS * _NUM_TOKENS))
            laux_ref[...] = jnp.sum(
                cnt1 * sgates[...], keepdims=True).reshape(1, 1) * scale

    @pl.when(i >= _NB1)
    def _expand():
        j = i - _NB1
        sl = pl.ds(j * _T2, _T2)
        pk1 = pk1_ref[sl, :]
        pk2 = pk2_ref[sl, :]
        g1 = g1_ref[sl, :]
        g2 = g2_ref[sl, :]
        cnt1 = carry1[...]

        idx1 = jnp.floor(pk1 * (1.0 / 4096.0))
        idx2 = jnp.floor(pk2 * (1.0 / 4096.0))
        loc1 = pk1 - idx1 * 4096.0
        loc2p = pk2 - idx2 * 4096.0

        iota_e = lax.broadcasted_iota(
            jnp.int32, (_T2, _NUM_EXPERTS), 1).astype(jnp.float32)
        onehot1 = (iota_e == idx1).astype(jnp.float32)
        onehot2 = (iota_e == idx2).astype(jnp.float32)
        loc2 = loc2p + jnp.sum(cnt1 * onehot2, axis=1, keepdims=True)

        cap = jnp.float32(_CAPACITY)
        w1 = jnp.where(loc1 < cap, g1, 0.0)
        w2 = jnp.where(loc2 < cap, g2, 0.0)
        denom = jnp.maximum(w1 + w2, _EPS)
        w1 = w1 / denom
        w2 = w2 / denom

        b = onehot1 * w1 + onehot2 * w2
        l = onehot1 * loc1 + onehot2 * loc2 + (1.0 - onehot1 - onehot2) * -1.0
        l = jnp.where(b != 0.0, l, -1.0)
        pk = l + b * 0.5
        pkb = pk[:, :, None]
        fl = jnp.floor(pkb)
        iota_c = lax.broadcasted_iota(
            jnp.int32, (_T2, _NUM_EXPERTS, _CAPACITY), 2).astype(jnp.float32)
        hit = iota_c == fl
        comb_ref[...] = jnp.where(hit, (pkb - fl) * 2.0, 0.0)
        mask_ref[...] = hit


@jax.jit
def kernel(input, wg):
    n = input.shape[0]

    gate = pl.pallas_call(
        _gate_kernel,
        grid=(_NB1 + _NB2,),
        in_specs=[
            pl.BlockSpec((_T1, _MODEL_DIM),
                         lambda i: (jnp.minimum(i, _NB1 - 1), 0)),
            pl.BlockSpec((_MODEL_DIM, _NUM_EXPERTS), lambda i: (0, 0)),
        ],
        out_specs=[
            pl.BlockSpec((_T2, _NUM_EXPERTS, _CAPACITY),
                         lambda i: (jnp.maximum(i - _NB1, 0), 0, 0)),
            pl.BlockSpec((_T2, _NUM_EXPERTS, _CAPACITY),
                         lambda i: (jnp.maximum(i - _NB1, 0), 0, 0)),
            pl.BlockSpec((1, 1), lambda i: (0, 0)),
        ],
        out_shape=[
            jax.ShapeDtypeStruct((n, _NUM_EXPERTS, _CAPACITY), jnp.float32),
            jax.ShapeDtypeStruct((n, _NUM_EXPERTS, _CAPACITY), jnp.bool_),
            jax.ShapeDtypeStruct((1, 1), jnp.float32),
        ],
        scratch_shapes=[
            pltpu.VMEM((_NUM_TOKENS, 1), jnp.float32),
            pltpu.VMEM((_NUM_TOKENS, 1), jnp.float32),
            pltpu.VMEM((_NUM_TOKENS, 1), jnp.float32),
            pltpu.VMEM((_NUM_TOKENS, 1), jnp.float32),
            pltpu.VMEM((1, _NUM_EXPERTS), jnp.float32),
            pltpu.VMEM((1, _NUM_EXPERTS), jnp.float32),
            pltpu.VMEM((1, _NUM_EXPERTS), jnp.float32),
        ],
    )
    combine_weights, dispatch_mask, laux = gate(input, wg)
    l_aux = laux.reshape(())
    return (l_aux, combine_weights, dispatch_mask)
